# initial kernel scaffold (unmeasured)
import jax
import jax.numpy as jnp
from jax import lax
from jax.experimental import pallas as pl
from jax.experimental.pallas import tpu as pltpu


def kernel(
    x,
):
    def body(*refs):
        pass

    out_shape = jax.ShapeDtypeStruct(..., jnp.float32)
    return pl.pallas_call(body, out_shape=out_shape)(...)


# baseline (device time: 1068029 ns/iter reference)
import jax
import jax.numpy as jnp
from jax import lax
from jax.experimental import pallas as pl
from jax.experimental.pallas import tpu as pltpu


def kernel(x):
    m_per, n = x.shape

    def body(x_ref, out_ref, local_sem, send_sem, recv_sem):
        my_x = lax.axis_index("x")
        my_y = lax.axis_index("y")
        my_z = lax.axis_index("z")
        partner = (1 - my_x, my_y, my_z)

        barrier_sem = pltpu.get_barrier_semaphore()
        pl.semaphore_signal(
            barrier_sem, inc=1,
            device_id=partner, device_id_type=pl.DeviceIdType.MESH,
        )
        pl.semaphore_wait(barrier_sem, 1)

        local = pltpu.make_async_copy(
            x_ref, out_ref.at[pl.ds(my_x * m_per, m_per)], local_sem
        )
        local.start()

        rdma = pltpu.make_async_remote_copy(
            src_ref=x_ref,
            dst_ref=out_ref.at[pl.ds(my_x * m_per, m_per)],
            send_sem=send_sem,
            recv_sem=recv_sem,
            device_id=partner,
            device_id_type=pl.DeviceIdType.MESH,
        )
        rdma.start()

        local.wait()
        rdma.wait()

    return pl.pallas_call(
        body,
        out_shape=jax.ShapeDtypeStruct((2 * m_per, n), x.dtype),
        in_specs=[pl.BlockSpec(memory_space=pl.ANY)],
        out_specs=pl.BlockSpec(memory_space=pl.ANY),
        scratch_shapes=[
            pltpu.SemaphoreType.DMA,
            pltpu.SemaphoreType.DMA,
            pltpu.SemaphoreType.DMA,
        ],
        compiler_params=pltpu.CompilerParams(collective_id=0),
    )(x)


# device time: 212062 ns/iter; 5.0364x vs baseline; 5.0364x over previous
import jax
import jax.numpy as jnp
from jax import lax
from jax.experimental import pallas as pl
from jax.experimental.pallas import tpu as pltpu

Q_ROWS = 2048
CQ = 4
CH = Q_ROWS // CQ
HALF = CQ // 2
LB_CHUNKS = 8
LB_SLOTS = 4


def kernel(x):
    m_per, n = x.shape

    def body(
        x_ref,
        out_ref,
        lb_vmem,
        lb_h2v_sems,
        lb_v2h_sems,
        sx, rx,
        s_yd, r_yd,
        s_zd, r_zd,
        s_ydg, r_ydg,
        s_zdg, r_zdg,
    ):
        my_x = lax.axis_index("x")
        my_y = lax.axis_index("y")
        my_z = lax.axis_index("z")
        p_dev = (1 - my_x, my_y, my_z)
        yn_dev = (my_x, 1 - my_y, my_z)
        zn_dev = (my_x, my_y, 1 - my_z)

        mybase = my_x * m_per
        pbase = (1 - my_x) * m_per

        q_me = 2 * my_y + my_z
        q_yn = 2 * (1 - my_y) + my_z
        q_zn = 2 * my_y + (1 - my_z)
        q_diag = 2 * (1 - my_y) + (1 - my_z)

        barrier_sem = pltpu.get_barrier_semaphore()
        for nbr in (p_dev, yn_dev, zn_dev):
            pl.semaphore_signal(
                barrier_sem, inc=1,
                device_id=nbr, device_id_type=pl.DeviceIdType.MESH,
            )
        pl.semaphore_wait(barrier_sem, 3)

        def remote(src, dst, send_sem, recv_sem, dev):
            return pltpu.make_async_remote_copy(
                src_ref=src, dst_ref=dst,
                send_sem=send_sem, recv_sem=recv_sem,
                device_id=dev, device_id_type=pl.DeviceIdType.MESH,
            )

        def out_rows(start):
            return out_ref.at[pl.ds(start, CH)]

        sends = []

        for c in range(CQ):
            r = remote(
                x_ref.at[pl.ds(q_me * Q_ROWS + c * CH, CH)],
                out_rows(mybase + q_me * Q_ROWS + c * CH),
                sx.at[c], rx.at[c], p_dev,
            )
            r.start()
            sends.append(r)

        lb_rows = m_per // LB_CHUNKS
        lb_h2v = []
        lb_v2h = []

        def lb_start_h2v(c):
            s = c % LB_SLOTS
            h = pltpu.make_async_copy(
                x_ref.at[pl.ds(c * lb_rows, lb_rows)], lb_vmem.at[s],
                lb_h2v_sems.at[s],
            )
            h.start()
            lb_h2v.append(h)

        def lb_start_v2h(c):
            s = c % LB_SLOTS
            v = pltpu.make_async_copy(
                lb_vmem.at[s],
                out_ref.at[pl.ds(mybase + c * lb_rows, lb_rows)],
                lb_v2h_sems.at[s],
            )
            v.start()
            lb_v2h.append(v)

        for c in range(LB_CHUNKS):
            if c >= LB_SLOTS:
                lb_v2h[c - LB_SLOTS].wait()
            lb_start_h2v(c)
            if c >= 1:
                lb_h2v[c - 1].wait()
                lb_start_v2h(c - 1)
        lb_h2v[LB_CHUNKS - 1].wait()
        lb_start_v2h(LB_CHUNKS - 1)

        def recv(dst, recv_sem):
            return remote(
                x_ref.at[pl.ds(0, CH)], dst, sx.at[0], recv_sem, p_dev
            )

        for c in range(CQ):
            off = pbase + q_me * Q_ROWS + c * CH
            recv(out_rows(off), rx.at[c]).wait_recv()
            r = remote(out_rows(off), out_rows(off), s_yd.at[c], r_yd.at[c], yn_dev)
            r.start()
            sends.append(r)
            r = remote(out_rows(off), out_rows(off), s_zd.at[c], r_zd.at[c], zn_dev)
            r.start()
            sends.append(r)

        for c in range(CQ):
            recv(out_rows(pbase + q_yn * Q_ROWS + c * CH), r_yd.at[c]).wait_recv()
            if c >= HALF:
                off = pbase + q_yn * Q_ROWS + c * CH
                r = remote(
                    out_rows(off), out_rows(off),
                    s_zdg.at[c - HALF], r_zdg.at[c - HALF], zn_dev,
                )
                r.start()
                sends.append(r)
            recv(out_rows(pbase + q_zn * Q_ROWS + c * CH), r_zd.at[c]).wait_recv()
            if c < HALF:
                off = pbase + q_zn * Q_ROWS + c * CH
                r = remote(
                    out_rows(off), out_rows(off),
                    s_ydg.at[c], r_ydg.at[c], yn_dev,
                )
                r.start()
                sends.append(r)

        for c in range(HALF):
            recv(out_rows(pbase + q_diag * Q_ROWS + c * CH), r_ydg.at[c]).wait_recv()
        for c in range(HALF):
            recv(
                out_rows(pbase + q_diag * Q_ROWS + (c + HALF) * CH),
                r_zdg.at[c],
            ).wait_recv()

        for c in range(max(0, LB_CHUNKS - LB_SLOTS), LB_CHUNKS):
            lb_v2h[c].wait()
        for r in sends:
            r.wait_send()

    lb_rows = m_per // LB_CHUNKS
    return pl.pallas_call(
        body,
        out_shape=jax.ShapeDtypeStruct((2 * m_per, n), x.dtype),
        in_specs=[pl.BlockSpec(memory_space=pl.ANY)],
        out_specs=pl.BlockSpec(memory_space=pl.ANY),
        scratch_shapes=[
            pltpu.VMEM((LB_SLOTS, lb_rows, n), jnp.float32),
            pltpu.SemaphoreType.DMA((LB_SLOTS,)),
            pltpu.SemaphoreType.DMA((LB_SLOTS,)),
            pltpu.SemaphoreType.DMA((CQ,)),
            pltpu.SemaphoreType.DMA((CQ,)),
            pltpu.SemaphoreType.DMA((CQ,)),
            pltpu.SemaphoreType.DMA((CQ,)),
            pltpu.SemaphoreType.DMA((CQ,)),
            pltpu.SemaphoreType.DMA((CQ,)),
            pltpu.SemaphoreType.DMA((HALF,)),
            pltpu.SemaphoreType.DMA((HALF,)),
            pltpu.SemaphoreType.DMA((HALF,)),
            pltpu.SemaphoreType.DMA((HALF,)),
        ],
        compiler_params=pltpu.CompilerParams(collective_id=0),
    )(x)


# device time: 196902 ns/iter; 5.4242x vs baseline; 1.0770x over previous
import jax
import jax.numpy as jnp
from jax import lax
from jax.experimental import pallas as pl
from jax.experimental.pallas import tpu as pltpu

Q_ROWS = 2048
CQ = 8
CH = Q_ROWS // CQ
X_DIAG = (0, 1, 2)
Y_DIAG = (3, 4, 5)
Z_DIAG = (6, 7)
LB_CHUNKS = 8
LB_SLOTS = 4


def kernel(x):
    m_per, n = x.shape

    def body(
        x_ref,
        out_ref,
        lb_vmem,
        lb_h2v_sems,
        lb_v2h_sems,
        sx, rx,
        sxd, rxd,
        s_yd, r_yd,
        s_zd, r_zd,
        s_ydg, r_ydg,
        s_zdg, r_zdg,
    ):
        my_x = lax.axis_index("x")
        my_y = lax.axis_index("y")
        my_z = lax.axis_index("z")
        p_dev = (1 - my_x, my_y, my_z)
        yn_dev = (my_x, 1 - my_y, my_z)
        zn_dev = (my_x, my_y, 1 - my_z)

        mybase = my_x * m_per
        pbase = (1 - my_x) * m_per

        q_me = 2 * my_y + my_z
        q_yn = 2 * (1 - my_y) + my_z
        q_zn = 2 * my_y + (1 - my_z)
        q_diag = 2 * (1 - my_y) + (1 - my_z)

        barrier_sem = pltpu.get_barrier_semaphore()
        for nbr in (p_dev, yn_dev, zn_dev):
            pl.semaphore_signal(
                barrier_sem, inc=1,
                device_id=nbr, device_id_type=pl.DeviceIdType.MESH,
            )
        pl.semaphore_wait(barrier_sem, 3)

        def remote(src, dst, send_sem, recv_sem, dev):
            return pltpu.make_async_remote_copy(
                src_ref=src, dst_ref=dst,
                send_sem=send_sem, recv_sem=recv_sem,
                device_id=dev, device_id_type=pl.DeviceIdType.MESH,
            )

        def out_rows(start):
            return out_ref.at[pl.ds(start, CH)]

        sends = []

        for c in range(CQ):
            r = remote(
                x_ref.at[pl.ds(q_me * Q_ROWS + c * CH, CH)],
                out_rows(mybase + q_me * Q_ROWS + c * CH),
                sx.at[c], rx.at[c], p_dev,
            )
            r.start()
            sends.append(r)
        for i, c in enumerate(X_DIAG):
            r = remote(
                x_ref.at[pl.ds(q_diag * Q_ROWS + c * CH, CH)],
                out_rows(mybase + q_diag * Q_ROWS + c * CH),
                sxd.at[i], rxd.at[i], p_dev,
            )
            r.start()
            sends.append(r)

        lb_rows = m_per // LB_CHUNKS
        lb_h2v = []
        lb_v2h = []

        def lb_start_h2v(c):
            s = c % LB_SLOTS
            h = pltpu.make_async_copy(
                x_ref.at[pl.ds(c * lb_rows, lb_rows)], lb_vmem.at[s],
                lb_h2v_sems.at[s],
            )
            h.start()
            lb_h2v.append(h)

        def lb_start_v2h(c):
            s = c % LB_SLOTS
            v = pltpu.make_async_copy(
                lb_vmem.at[s],
                out_ref.at[pl.ds(mybase + c * lb_rows, lb_rows)],
                lb_v2h_sems.at[s],
            )
            v.start()
            lb_v2h.append(v)

        for c in range(LB_CHUNKS):
            if c >= LB_SLOTS:
                lb_v2h[c - LB_SLOTS].wait()
            lb_start_h2v(c)
            if c >= 1:
                lb_h2v[c - 1].wait()
                lb_start_v2h(c - 1)
        lb_h2v[LB_CHUNKS - 1].wait()
        lb_start_v2h(LB_CHUNKS - 1)

        def recv(dst, recv_sem):
            return remote(
                x_ref.at[pl.ds(0, CH)], dst, sx.at[0], recv_sem, p_dev
            )

        for c in range(CQ):
            off = pbase + q_me * Q_ROWS + c * CH
            recv(out_rows(off), rx.at[c]).wait_recv()
            r = remote(out_rows(off), out_rows(off), s_yd.at[c], r_yd.at[c], yn_dev)
            r.start()
            sends.append(r)
            r = remote(out_rows(off), out_rows(off), s_zd.at[c], r_zd.at[c], zn_dev)
            r.start()
            sends.append(r)

        for c in range(CQ):
            recv(out_rows(pbase + q_yn * Q_ROWS + c * CH), r_yd.at[c]).wait_recv()
            if c in Z_DIAG:
                off = pbase + q_yn * Q_ROWS + c * CH
                i = Z_DIAG.index(c)
                r = remote(
                    out_rows(off), out_rows(off),
                    s_zdg.at[i], r_zdg.at[i], zn_dev,
                )
                r.start()
                sends.append(r)
            recv(out_rows(pbase + q_zn * Q_ROWS + c * CH), r_zd.at[c]).wait_recv()
            if c in Y_DIAG:
                off = pbase + q_zn * Q_ROWS + c * CH
                i = Y_DIAG.index(c)
                r = remote(
                    out_rows(off), out_rows(off),
                    s_ydg.at[i], r_ydg.at[i], yn_dev,
                )
                r.start()
                sends.append(r)

        for i, c in enumerate(X_DIAG):
            recv(out_rows(pbase + q_diag * Q_ROWS + c * CH), rxd.at[i]).wait_recv()
        for i, c in enumerate(Y_DIAG):
            recv(out_rows(pbase + q_diag * Q_ROWS + c * CH), r_ydg.at[i]).wait_recv()
        for i, c in enumerate(Z_DIAG):
            recv(out_rows(pbase + q_diag * Q_ROWS + c * CH), r_zdg.at[i]).wait_recv()

        for c in range(max(0, LB_CHUNKS - LB_SLOTS), LB_CHUNKS):
            lb_v2h[c].wait()
        for r in sends:
            r.wait_send()

    lb_rows = m_per // LB_CHUNKS
    return pl.pallas_call(
        body,
        out_shape=jax.ShapeDtypeStruct((2 * m_per, n), x.dtype),
        in_specs=[pl.BlockSpec(memory_space=pl.ANY)],
        out_specs=pl.BlockSpec(memory_space=pl.ANY),
        scratch_shapes=[
            pltpu.VMEM((LB_SLOTS, lb_rows, n), jnp.float32),
            pltpu.SemaphoreType.DMA((LB_SLOTS,)),
            pltpu.SemaphoreType.DMA((LB_SLOTS,)),
            pltpu.SemaphoreType.DMA((CQ,)),
            pltpu.SemaphoreType.DMA((CQ,)),
            pltpu.SemaphoreType.DMA((len(X_DIAG),)),
            pltpu.SemaphoreType.DMA((len(X_DIAG),)),
            pltpu.SemaphoreType.DMA((CQ,)),
            pltpu.SemaphoreType.DMA((CQ,)),
            pltpu.SemaphoreType.DMA((CQ,)),
            pltpu.SemaphoreType.DMA((CQ,)),
            pltpu.SemaphoreType.DMA((len(Y_DIAG),)),
            pltpu.SemaphoreType.DMA((len(Y_DIAG),)),
            pltpu.SemaphoreType.DMA((len(Z_DIAG),)),
            pltpu.SemaphoreType.DMA((len(Z_DIAG),)),
        ],
        compiler_params=pltpu.CompilerParams(collective_id=0),
    )(x)
